# baseline (speedup 1.0000x reference)
import functools

import jax
import jax.numpy as jnp
from jax import lax
from jax.experimental import pallas as pl
from jax.experimental.pallas import tpu as pltpu
from jax.experimental.pallas import tpu_sc as plsc

N = 10000
E = 320000
D_IN = 128
D_HID = 256
D_OUT = 128

NC = 2
NS = 16
NW = NC * NS
EPW = E // NW
B = 80
NB = EPW // B
BS = 64
EPWP = 10240
NBS = EPWP // BS
NCHK = 2
NBC = NBS // NCHK
NPAIR = NBC // 2
NPAD = 10240
RPT = NPAD // NS
RSTG = 128

_mesh = plsc.VectorSubcoreMesh(core_axis_name="c", subcore_axis_name="s")


def _zero_stage(stage_v, rows, width):
    nv = width // 16

    def body(i, _):
        for j in range(nv):
            stage_v[i, pl.ds(j * 16, 16)] = jnp.zeros((16,), jnp.float32)
        return 0

    lax.fori_loop(0, rows, body, 0)


@functools.partial(
    pl.kernel,
    out_type=jax.ShapeDtypeStruct((NC, NPAD, 128), jnp.float32),
    mesh=_mesh,
    scratch_types=[
        pltpu.VMEM((NB, B), jnp.int32),
        pltpu.VMEM((B, 128), jnp.float32),
        pltpu.VMEM_SHARED((NPAD, 128), jnp.float32),
    ],
)
def _sc_degree(dst_hbm, out_hbm, dst_v, ones_v, acc_sh):
    c = lax.axis_index("c")
    s = lax.axis_index("s")
    wid = s * NC + c

    _zero_stage(ones_v, B, 128)
    for t in range(RPT // B):
        pltpu.sync_copy(ones_v, acc_sh.at[pl.ds(s * RPT + t * B, B)])
    plsc.subcore_barrier()

    def fill_ones(i, _):
        for j in range(8):
            ones_v[i, pl.ds(j * 16, 16)] = jnp.ones((16,), jnp.float32)
        return 0

    lax.fori_loop(0, B, fill_ones, 0)

    pltpu.sync_copy(dst_hbm.at[wid], dst_v)

    def body(j, _):
        pltpu.sync_copy(ones_v, acc_sh.at[dst_v.at[j]], add=True)
        return 0

    lax.fori_loop(0, NB, body, 0)
    plsc.subcore_barrier()

    for t in range(RPT // B):
        r0 = s * RPT + t * B
        pltpu.sync_copy(acc_sh.at[pl.ds(r0, B)], ones_v)
        pltpu.sync_copy(ones_v, out_hbm.at[c, pl.ds(r0, B)])


@functools.partial(
    pl.kernel,
    out_type=jax.ShapeDtypeStruct((NC, NPAD, 128), jnp.float32),
    mesh=_mesh,
    scratch_types=[
        pltpu.VMEM((NBC, BS), jnp.int32),
        pltpu.VMEM((NBC, BS), jnp.int32),
        pltpu.VMEM((BS, 128), jnp.float32),
        pltpu.VMEM((BS, 128), jnp.float32),
        pltpu.VMEM_SHARED((NPAD, 128), jnp.float32),
        pltpu.SemaphoreType.DMA,
        pltpu.SemaphoreType.DMA,
    ],
)
def _sc_spmm(xs_hbm, src_hbm, dst_hbm, out_hbm,
             src_v, dst_v, rows0_v, rows1_v, acc_sh, sem0, sem1):
    c = lax.axis_index("c")
    s = lax.axis_index("s")
    wid = s * NC + c

    _zero_stage(rows0_v, BS, 128)
    for t in range(RPT // BS):
        pltpu.sync_copy(rows0_v, acc_sh.at[pl.ds(s * RPT + t * BS, BS)])
    plsc.subcore_barrier()

    def pair(t, _):
        j0 = 2 * t
        d0 = pltpu.async_copy(xs_hbm.at[src_v.at[j0]], rows0_v, sem0)
        d0.wait()
        pltpu.sync_copy(rows0_v, acc_sh.at[dst_v.at[j0]], add=True)
        d1 = pltpu.async_copy(xs_hbm.at[src_v.at[j0 + 1]], rows1_v, sem1)
        d1.wait()
        pltpu.sync_copy(rows1_v, acc_sh.at[dst_v.at[j0 + 1]], add=True)
        return 0

    for chk in range(NCHK):
        pltpu.sync_copy(src_hbm.at[wid, pl.ds(chk * NBC, NBC)], src_v)
        pltpu.sync_copy(dst_hbm.at[wid, pl.ds(chk * NBC, NBC)], dst_v)
        lax.fori_loop(0, NPAIR, pair, 0)
    plsc.subcore_barrier()

    for t in range(RPT // BS):
        r0 = s * RPT + t * BS
        pltpu.sync_copy(acc_sh.at[pl.ds(r0, BS)], rows0_v)
        pltpu.sync_copy(rows0_v, out_hbm.at[c, pl.ds(r0, BS)])


_BR = 400
_GRID = N // _BR


def _tc_pre_body(degp_ref, x_ref, dis_ref, xs1_ref):
    deg = degp_ref[0, :, 0:1] + degp_ref[1, :, 0:1] + 1.0
    dis = lax.rsqrt(deg)
    dis_ref[...] = dis
    xs1_ref[...] = x_ref[...] * dis


def _tc_pre(degp, x):
    return pl.pallas_call(
        _tc_pre_body,
        grid=(_GRID,),
        in_specs=[
            pl.BlockSpec((NC, _BR, 128), lambda i: (0, i, 0)),
            pl.BlockSpec((_BR, D_IN), lambda i: (i, 0)),
        ],
        out_specs=[
            pl.BlockSpec((_BR, 1), lambda i: (i, 0)),
            pl.BlockSpec((_BR, D_IN), lambda i: (i, 0)),
        ],
        out_shape=[
            jax.ShapeDtypeStruct((N, 1), jnp.float32),
            jax.ShapeDtypeStruct((N, D_IN), jnp.float32),
        ],
    )(degp, x)


def _tc_layer1_body(s1_ref, xs1_ref, dis_ref, w1_ref, b1_ref, xs2_ref):
    dis = dis_ref[...]
    agg = dis * (s1_ref[0] + s1_ref[1] + xs1_ref[...])
    y1 = jnp.maximum(
        jnp.dot(agg, w1_ref[...], preferred_element_type=jnp.float32)
        + b1_ref[...], 0.0)
    xs2_ref[...] = dis * y1


def _tc_layer1(s1, xs1, dis, W1, b1):
    return pl.pallas_call(
        _tc_layer1_body,
        grid=(_GRID,),
        in_specs=[
            pl.BlockSpec((NC, _BR, D_IN), lambda i: (0, i, 0)),
            pl.BlockSpec((_BR, D_IN), lambda i: (i, 0)),
            pl.BlockSpec((_BR, 1), lambda i: (i, 0)),
            pl.BlockSpec((D_IN, D_HID), lambda i: (0, 0)),
            pl.BlockSpec((1, D_HID), lambda i: (0, 0)),
        ],
        out_specs=pl.BlockSpec((_BR, D_HID), lambda i: (i, 0)),
        out_shape=jax.ShapeDtypeStruct((N, D_HID), jnp.float32),
    )(s1, xs1, dis, W1, b1.reshape(1, D_HID))


def _tc_layer2_body(s2a_ref, s2b_ref, xs2_ref, dis_ref, w2_ref, b2_ref,
                    w3_ref, xs3_ref):
    dis = dis_ref[...]
    xs2 = xs2_ref[...]
    agg_a = dis * (s2a_ref[0] + s2a_ref[1] + xs2[:, :D_IN])
    agg_b = dis * (s2b_ref[0] + s2b_ref[1] + xs2[:, D_IN:])
    agg = jnp.concatenate([agg_a, agg_b], axis=1)
    y2 = jnp.maximum(
        jnp.dot(agg, w2_ref[...], preferred_element_type=jnp.float32)
        + b2_ref[...], 0.0)
    z = jnp.dot(y2, w3_ref[...], preferred_element_type=jnp.float32)
    xs3_ref[...] = dis * z


def _tc_layer2(s2a, s2b, xs2, dis, W2, b2, W3):
    return pl.pallas_call(
        _tc_layer2_body,
        grid=(_GRID,),
        in_specs=[
            pl.BlockSpec((NC, _BR, D_IN), lambda i: (0, i, 0)),
            pl.BlockSpec((NC, _BR, D_IN), lambda i: (0, i, 0)),
            pl.BlockSpec((_BR, D_HID), lambda i: (i, 0)),
            pl.BlockSpec((_BR, 1), lambda i: (i, 0)),
            pl.BlockSpec((D_HID, D_HID), lambda i: (0, 0)),
            pl.BlockSpec((1, D_HID), lambda i: (0, 0)),
            pl.BlockSpec((D_HID, D_OUT), lambda i: (0, 0)),
        ],
        out_specs=pl.BlockSpec((_BR, D_OUT), lambda i: (i, 0)),
        out_shape=jax.ShapeDtypeStruct((N, D_OUT), jnp.float32),
    )(s2a, s2b, xs2, dis, W2, b2.reshape(1, D_HID), W3)


def _tc_final_body(s3_ref, xs3_ref, dis_ref, b3_ref, out_ref):
    dis = dis_ref[...]
    out_ref[...] = dis * (s3_ref[0] + s3_ref[1] + xs3_ref[...]) + b3_ref[...]


def _tc_final(s3, xs3, dis, b3):
    return pl.pallas_call(
        _tc_final_body,
        grid=(_GRID,),
        in_specs=[
            pl.BlockSpec((NC, _BR, D_OUT), lambda i: (0, i, 0)),
            pl.BlockSpec((_BR, D_OUT), lambda i: (i, 0)),
            pl.BlockSpec((_BR, 1), lambda i: (i, 0)),
            pl.BlockSpec((1, D_OUT), lambda i: (0, 0)),
        ],
        out_specs=pl.BlockSpec((_BR, D_OUT), lambda i: (i, 0)),
        out_shape=jax.ShapeDtypeStruct((N, D_OUT), jnp.float32),
    )(s3, xs3, dis, b3.reshape(1, D_OUT))


def kernel(x, edge_index, W1, b1, W2, b2, W3, b3):
    npad_e = NW * EPWP - E
    src = jnp.concatenate(
        [edge_index[0], jnp.zeros((npad_e,), jnp.int32)]).reshape(NW, NBS, BS)
    pad_dst = N + (jnp.arange(npad_e, dtype=jnp.int32) % (NPAD - N))
    dst = jnp.concatenate([edge_index[1], pad_dst]).reshape(NW, NBS, BS)
    dst_deg = edge_index[1].reshape(NW, NB, B)

    degp = _sc_degree(dst_deg)
    dis, xs1 = _tc_pre(degp, x)

    s1 = _sc_spmm(xs1, src, dst)
    xs2 = _tc_layer1(s1, xs1, dis, W1, b1)

    s2a = _sc_spmm(xs2[:, :D_IN], src, dst)
    s2b = _sc_spmm(xs2[:, D_IN:], src, dst)
    xs3 = _tc_layer2(s2a, s2b, xs2, dis, W2, b2, W3)

    s3 = _sc_spmm(xs3, src, dst)
    return _tc_final(s3, xs3, dis, b3)


# speedup vs baseline: 1.0806x; 1.0806x over previous
import functools

import jax
import jax.numpy as jnp
from jax import lax
from jax.experimental import pallas as pl
from jax.experimental.pallas import tpu as pltpu
from jax.experimental.pallas import tpu_sc as plsc

N = 10000
E = 320000
D_IN = 128
D_HID = 256
D_OUT = 128

NC = 2
NS = 16
NW = NC * NS
EPW = E // NW
B = 80
NB = EPW // B
BS = 64
EPWP = 10240
NBS = EPWP // BS
NCHK = 2
NBC = NBS // NCHK
NPAIR = NBC // 2
NPAD = 10240
RPT = NPAD // NS
RSTG = 128

_mesh = plsc.VectorSubcoreMesh(core_axis_name="c", subcore_axis_name="s")


def _zero_stage(stage_v, rows, width):
    nv = width // 16

    def body(i, _):
        for j in range(nv):
            stage_v[i, pl.ds(j * 16, 16)] = jnp.zeros((16,), jnp.float32)
        return 0

    lax.fori_loop(0, rows, body, 0)


@functools.partial(
    pl.kernel,
    out_type=jax.ShapeDtypeStruct((NC, NPAD, 128), jnp.float32),
    mesh=_mesh,
    scratch_types=[
        pltpu.VMEM((NB, B), jnp.int32),
        pltpu.VMEM((B, 128), jnp.float32),
        pltpu.VMEM_SHARED((NPAD, 128), jnp.float32),
    ],
)
def _sc_degree(dst_hbm, out_hbm, dst_v, ones_v, acc_sh):
    c = lax.axis_index("c")
    s = lax.axis_index("s")
    wid = s * NC + c

    _zero_stage(ones_v, B, 128)
    for t in range(RPT // B):
        pltpu.sync_copy(ones_v, acc_sh.at[pl.ds(s * RPT + t * B, B)])
    plsc.subcore_barrier()

    def fill_ones(i, _):
        for j in range(8):
            ones_v[i, pl.ds(j * 16, 16)] = jnp.ones((16,), jnp.float32)
        return 0

    lax.fori_loop(0, B, fill_ones, 0)

    pltpu.sync_copy(dst_hbm.at[wid], dst_v)

    def body(j, _):
        pltpu.sync_copy(ones_v, acc_sh.at[dst_v.at[j]], add=True)
        return 0

    lax.fori_loop(0, NB, body, 0)
    plsc.subcore_barrier()

    for t in range(RPT // B):
        r0 = s * RPT + t * B
        pltpu.sync_copy(acc_sh.at[pl.ds(r0, B)], ones_v)
        pltpu.sync_copy(ones_v, out_hbm.at[c, pl.ds(r0, B)])


@functools.partial(
    pl.kernel,
    out_type=jax.ShapeDtypeStruct((NC, NPAD, 128), jnp.float32),
    mesh=_mesh,
    scratch_types=[
        pltpu.VMEM((NBC, BS), jnp.int32),
        pltpu.VMEM((NBC, BS), jnp.int32),
        pltpu.VMEM((BS, 128), jnp.float32),
        pltpu.VMEM((BS, 128), jnp.float32),
        pltpu.VMEM_SHARED((NPAD, 128), jnp.float32),
        pltpu.SemaphoreType.DMA,
        pltpu.SemaphoreType.DMA,
    ],
)
def _sc_spmm(xs_hbm, src_hbm, dst_hbm, out_hbm,
             src_v, dst_v, rows0_v, rows1_v, acc_sh, sem0, sem1):
    c = lax.axis_index("c")
    s = lax.axis_index("s")
    wid = s * NC + c

    _zero_stage(rows0_v, BS, 128)
    for t in range(RPT // BS):
        pltpu.sync_copy(rows0_v, acc_sh.at[pl.ds(s * RPT + t * BS, BS)])
    plsc.subcore_barrier()

    def pair(t, _):
        j0 = 2 * t
        d0 = pltpu.async_copy(xs_hbm.at[src_v.at[j0]], rows0_v, sem0)
        d0.wait()
        pltpu.sync_copy(rows0_v, acc_sh.at[dst_v.at[j0]], add=True)
        d1 = pltpu.async_copy(xs_hbm.at[src_v.at[j0 + 1]], rows1_v, sem1)
        d1.wait()
        pltpu.sync_copy(rows1_v, acc_sh.at[dst_v.at[j0 + 1]], add=True)
        return 0

    for chk in range(NCHK):
        pltpu.sync_copy(src_hbm.at[wid, pl.ds(chk * NBC, NBC)], src_v)
        pltpu.sync_copy(dst_hbm.at[wid, pl.ds(chk * NBC, NBC)], dst_v)
        lax.fori_loop(0, NPAIR, pair, 0)
    plsc.subcore_barrier()

    for t in range(RPT // BS):
        r0 = s * RPT + t * BS
        pltpu.sync_copy(acc_sh.at[pl.ds(r0, BS)], rows0_v)
        pltpu.sync_copy(rows0_v, out_hbm.at[c, pl.ds(r0, BS)])


_BR = 400
_GRID = N // _BR


def _tc_pre_body(degp_ref, x_ref, dis_ref, xs1_ref):
    deg = degp_ref[0, :, 0:1] + degp_ref[1, :, 0:1] + 1.0
    dis = lax.rsqrt(deg)
    dis_ref[...] = dis
    xs1_ref[...] = x_ref[...] * dis


def _tc_pre(degp, x):
    return pl.pallas_call(
        _tc_pre_body,
        grid=(_GRID,),
        in_specs=[
            pl.BlockSpec((NC, _BR, 128), lambda i: (0, i, 0)),
            pl.BlockSpec((_BR, D_IN), lambda i: (i, 0)),
        ],
        out_specs=[
            pl.BlockSpec((_BR, 1), lambda i: (i, 0)),
            pl.BlockSpec((_BR, D_IN), lambda i: (i, 0)),
        ],
        out_shape=[
            jax.ShapeDtypeStruct((N, 1), jnp.float32),
            jax.ShapeDtypeStruct((N, D_IN), jnp.float32),
        ],
    )(degp, x)


def _tc_layer1_body(s1_ref, xs1_ref, dis_ref, w1_ref, b1_ref, xs2_ref):
    dis = dis_ref[...]
    agg = dis * (s1_ref[0] + s1_ref[1] + xs1_ref[...])
    y1 = jnp.maximum(
        jnp.dot(agg, w1_ref[...], preferred_element_type=jnp.float32)
        + b1_ref[...], 0.0)
    xs2_ref[...] = dis * y1


def _tc_layer1(s1, xs1, dis, W1, b1):
    return pl.pallas_call(
        _tc_layer1_body,
        grid=(_GRID,),
        in_specs=[
            pl.BlockSpec((NC, _BR, D_IN), lambda i: (0, i, 0)),
            pl.BlockSpec((_BR, D_IN), lambda i: (i, 0)),
            pl.BlockSpec((_BR, 1), lambda i: (i, 0)),
            pl.BlockSpec((D_IN, D_HID), lambda i: (0, 0)),
            pl.BlockSpec((1, D_HID), lambda i: (0, 0)),
        ],
        out_specs=pl.BlockSpec((_BR, D_HID), lambda i: (i, 0)),
        out_shape=jax.ShapeDtypeStruct((N, D_HID), jnp.float32),
    )(s1, xs1, dis, W1, b1.reshape(1, D_HID))


def _tc_layer2_body(s2a_ref, s2b_ref, xs2_ref, dis_ref, w2_ref, b2_ref,
                    w3_ref, xs3_ref):
    dis = dis_ref[...]
    xs2 = xs2_ref[...]
    agg_a = dis * (s2a_ref[0] + s2a_ref[1] + xs2[:, :D_IN])
    agg_b = dis * (s2b_ref[0] + s2b_ref[1] + xs2[:, D_IN:])
    agg = jnp.concatenate([agg_a, agg_b], axis=1)
    y2 = jnp.maximum(
        jnp.dot(agg, w2_ref[...], preferred_element_type=jnp.float32)
        + b2_ref[...], 0.0)
    z = jnp.dot(y2, w3_ref[...], preferred_element_type=jnp.float32)
    xs3_ref[...] = dis * z


def _tc_layer2(s2a, s2b, xs2, dis, W2, b2, W3):
    return pl.pallas_call(
        _tc_layer2_body,
        grid=(_GRID,),
        in_specs=[
            pl.BlockSpec((NC, _BR, D_IN), lambda i: (0, i, 0)),
            pl.BlockSpec((NC, _BR, D_IN), lambda i: (0, i, 0)),
            pl.BlockSpec((_BR, D_HID), lambda i: (i, 0)),
            pl.BlockSpec((_BR, 1), lambda i: (i, 0)),
            pl.BlockSpec((D_HID, D_HID), lambda i: (0, 0)),
            pl.BlockSpec((1, D_HID), lambda i: (0, 0)),
            pl.BlockSpec((D_HID, D_OUT), lambda i: (0, 0)),
        ],
        out_specs=pl.BlockSpec((_BR, D_OUT), lambda i: (i, 0)),
        out_shape=jax.ShapeDtypeStruct((N, D_OUT), jnp.float32),
    )(s2a, s2b, xs2, dis, W2, b2.reshape(1, D_HID), W3)


def _tc_final_body(s3_ref, xs3_ref, dis_ref, b3_ref, out_ref):
    dis = dis_ref[...]
    out_ref[...] = dis * (s3_ref[0] + s3_ref[1] + xs3_ref[...]) + b3_ref[...]


def _tc_final(s3, xs3, dis, b3):
    return pl.pallas_call(
        _tc_final_body,
        grid=(_GRID,),
        in_specs=[
            pl.BlockSpec((NC, _BR, D_OUT), lambda i: (0, i, 0)),
            pl.BlockSpec((_BR, D_OUT), lambda i: (i, 0)),
            pl.BlockSpec((_BR, 1), lambda i: (i, 0)),
            pl.BlockSpec((1, D_OUT), lambda i: (0, 0)),
        ],
        out_specs=pl.BlockSpec((_BR, D_OUT), lambda i: (i, 0)),
        out_shape=jax.ShapeDtypeStruct((N, D_OUT), jnp.float32),
    )(s3, xs3, dis, b3.reshape(1, D_OUT))


def kernel(x, edge_index, W1, b1, W2, b2, W3, b3):
    ppw = EPWP - EPW
    pad_src = jnp.zeros((NW, ppw), jnp.int32)
    pad_dst = jnp.broadcast_to(
        N + jnp.arange(ppw, dtype=jnp.int32), (NW, ppw))
    src = jnp.concatenate(
        [edge_index[0].reshape(NW, EPW), pad_src], axis=1).reshape(NW, NBS, BS)
    dst = jnp.concatenate(
        [edge_index[1].reshape(NW, EPW), pad_dst], axis=1).reshape(NW, NBS, BS)
    dst_deg = edge_index[1].reshape(NW, NB, B)

    degp = _sc_degree(dst_deg)
    dis, xs1 = _tc_pre(degp, x)

    s1 = _sc_spmm(xs1, src, dst)
    xs2 = _tc_layer1(s1, xs1, dis, W1, b1)

    s2a = _sc_spmm(xs2[:, :D_IN], src, dst)
    s2b = _sc_spmm(xs2[:, D_IN:], src, dst)
    xs3 = _tc_layer2(s2a, s2b, xs2, dis, W2, b2, W3)

    s3 = _sc_spmm(xs3, src, dst)
    return _tc_final(s3, xs3, dis, b3)


# speedup vs baseline: 2.3705x; 2.1936x over previous
import functools

import jax
import jax.numpy as jnp
from jax import lax
from jax.experimental import pallas as pl
from jax.experimental.pallas import tpu as pltpu
from jax.experimental.pallas import tpu_sc as plsc

N = 10000
E = 320000
D_IN = 128
D_HID = 256
D_OUT = 128

NC = 2
NS = 16
NW = NC * NS
EPW = E // NW
B = 80
NB = EPW // B
BS = 64
EPWP = 10240
NBS = EPWP // BS
NCHK = 2
NBC = NBS // NCHK
NPAIR = NBC // 2
NPAD = 10240
RPT = NPAD // NS
RSTG = 128

_mesh = plsc.VectorSubcoreMesh(core_axis_name="c", subcore_axis_name="s")


def _zero_stage(stage_v, rows, width):
    nv = width // 16

    def body(i, _):
        for j in range(nv):
            stage_v[i, pl.ds(j * 16, 16)] = jnp.zeros((16,), jnp.float32)
        return 0

    lax.fori_loop(0, rows, body, 0)


@functools.partial(
    pl.kernel,
    out_type=jax.ShapeDtypeStruct((NC, NPAD, 128), jnp.float32),
    mesh=_mesh,
    scratch_types=[
        pltpu.VMEM((NB, B), jnp.int32),
        pltpu.VMEM((B, 128), jnp.float32),
        pltpu.VMEM_SHARED((NPAD, 128), jnp.float32),
    ],
)
def _sc_degree(dst_hbm, out_hbm, dst_v, ones_v, acc_sh):
    c = lax.axis_index("c")
    s = lax.axis_index("s")
    wid = s * NC + c

    _zero_stage(ones_v, B, 128)
    for t in range(RPT // B):
        pltpu.sync_copy(ones_v, acc_sh.at[pl.ds(s * RPT + t * B, B)])
    plsc.subcore_barrier()

    def fill_ones(i, _):
        for j in range(8):
            ones_v[i, pl.ds(j * 16, 16)] = jnp.ones((16,), jnp.float32)
        return 0

    lax.fori_loop(0, B, fill_ones, 0)

    pltpu.sync_copy(dst_hbm.at[wid], dst_v)

    def body(j, _):
        pltpu.sync_copy(ones_v, acc_sh.at[dst_v.at[j]], add=True)
        return 0

    lax.fori_loop(0, NB, body, 0)
    plsc.subcore_barrier()

    for t in range(RPT // B):
        r0 = s * RPT + t * B
        pltpu.sync_copy(acc_sh.at[pl.ds(r0, B)], ones_v)
        pltpu.sync_copy(ones_v, out_hbm.at[c, pl.ds(r0, B)])


@functools.partial(
    pl.kernel,
    out_type=jax.ShapeDtypeStruct((NC, NPAD, 128), jnp.float32),
    mesh=_mesh,
    scratch_types=[
        pltpu.VMEM((NB, B), jnp.int32),
        pltpu.VMEM((NB, B), jnp.int32),
        pltpu.VMEM((B, 128), jnp.float32),
        pltpu.VMEM_SHARED((NPAD, 128), jnp.float32),
        pltpu.SemaphoreType.DMA,
    ],
)
def _sc_spmm(xs_hbm, src_hbm, dst_hbm, out_hbm,
             src_v, dst_v, rows_v, acc_sh, sem):
    c = lax.axis_index("c")
    s = lax.axis_index("s")
    wid = s * NC + c

    _zero_stage(rows_v, B, 128)
    for t in range(RPT // B):
        pltpu.sync_copy(rows_v, acc_sh.at[pl.ds(s * RPT + t * B, B)])
    plsc.subcore_barrier()

    pltpu.sync_copy(src_hbm.at[wid], src_v)
    pltpu.sync_copy(dst_hbm.at[wid], dst_v)

    def body(j, _):
        pltpu.async_copy(xs_hbm.at[src_v.at[j]], rows_v, sem).wait()
        pltpu.sync_copy(rows_v, acc_sh.at[dst_v.at[j]], add=True)
        return 0

    lax.fori_loop(0, NB, body, 0)
    plsc.subcore_barrier()

    for t in range(RPT // B):
        r0 = s * RPT + t * B
        pltpu.sync_copy(acc_sh.at[pl.ds(r0, B)], rows_v)
        pltpu.sync_copy(rows_v, out_hbm.at[c, pl.ds(r0, B)])


_BR = 400
_GRID = N // _BR


def _tc_pre_body(degp_ref, x_ref, dis_ref, xs1_ref):
    deg = degp_ref[0, :, 0:1] + degp_ref[1, :, 0:1] + 1.0
    dis = lax.rsqrt(deg)
    dis_ref[...] = dis
    xs1_ref[...] = x_ref[...] * dis


def _tc_pre(degp, x):
    return pl.pallas_call(
        _tc_pre_body,
        grid=(_GRID,),
        in_specs=[
            pl.BlockSpec((NC, _BR, 128), lambda i: (0, i, 0)),
            pl.BlockSpec((_BR, D_IN), lambda i: (i, 0)),
        ],
        out_specs=[
            pl.BlockSpec((_BR, 1), lambda i: (i, 0)),
            pl.BlockSpec((_BR, D_IN), lambda i: (i, 0)),
        ],
        out_shape=[
            jax.ShapeDtypeStruct((N, 1), jnp.float32),
            jax.ShapeDtypeStruct((N, D_IN), jnp.float32),
        ],
    )(degp, x)


def _tc_layer1_body(s1_ref, xs1_ref, dis_ref, w1_ref, b1_ref, xs2_ref):
    dis = dis_ref[...]
    agg = dis * (s1_ref[0] + s1_ref[1] + xs1_ref[...])
    y1 = jnp.maximum(
        jnp.dot(agg, w1_ref[...], preferred_element_type=jnp.float32)
        + b1_ref[...], 0.0)
    xs2_ref[...] = dis * y1


def _tc_layer1(s1, xs1, dis, W1, b1):
    return pl.pallas_call(
        _tc_layer1_body,
        grid=(_GRID,),
        in_specs=[
            pl.BlockSpec((NC, _BR, D_IN), lambda i: (0, i, 0)),
            pl.BlockSpec((_BR, D_IN), lambda i: (i, 0)),
            pl.BlockSpec((_BR, 1), lambda i: (i, 0)),
            pl.BlockSpec((D_IN, D_HID), lambda i: (0, 0)),
            pl.BlockSpec((1, D_HID), lambda i: (0, 0)),
        ],
        out_specs=pl.BlockSpec((_BR, D_HID), lambda i: (i, 0)),
        out_shape=jax.ShapeDtypeStruct((N, D_HID), jnp.float32),
    )(s1, xs1, dis, W1, b1.reshape(1, D_HID))


def _tc_layer2_body(s2a_ref, s2b_ref, xs2_ref, dis_ref, w2_ref, b2_ref,
                    w3_ref, xs3_ref):
    dis = dis_ref[...]
    xs2 = xs2_ref[...]
    agg_a = dis * (s2a_ref[0] + s2a_ref[1] + xs2[:, :D_IN])
    agg_b = dis * (s2b_ref[0] + s2b_ref[1] + xs2[:, D_IN:])
    agg = jnp.concatenate([agg_a, agg_b], axis=1)
    y2 = jnp.maximum(
        jnp.dot(agg, w2_ref[...], preferred_element_type=jnp.float32)
        + b2_ref[...], 0.0)
    z = jnp.dot(y2, w3_ref[...], preferred_element_type=jnp.float32)
    xs3_ref[...] = dis * z


def _tc_layer2(s2a, s2b, xs2, dis, W2, b2, W3):
    return pl.pallas_call(
        _tc_layer2_body,
        grid=(_GRID,),
        in_specs=[
            pl.BlockSpec((NC, _BR, D_IN), lambda i: (0, i, 0)),
            pl.BlockSpec((NC, _BR, D_IN), lambda i: (0, i, 0)),
            pl.BlockSpec((_BR, D_HID), lambda i: (i, 0)),
            pl.BlockSpec((_BR, 1), lambda i: (i, 0)),
            pl.BlockSpec((D_HID, D_HID), lambda i: (0, 0)),
            pl.BlockSpec((1, D_HID), lambda i: (0, 0)),
            pl.BlockSpec((D_HID, D_OUT), lambda i: (0, 0)),
        ],
        out_specs=pl.BlockSpec((_BR, D_OUT), lambda i: (i, 0)),
        out_shape=jax.ShapeDtypeStruct((N, D_OUT), jnp.float32),
    )(s2a, s2b, xs2, dis, W2, b2.reshape(1, D_HID), W3)


def _tc_final_body(s3_ref, xs3_ref, dis_ref, b3_ref, out_ref):
    dis = dis_ref[...]
    out_ref[...] = dis * (s3_ref[0] + s3_ref[1] + xs3_ref[...]) + b3_ref[...]


def _tc_final(s3, xs3, dis, b3):
    return pl.pallas_call(
        _tc_final_body,
        grid=(_GRID,),
        in_specs=[
            pl.BlockSpec((NC, _BR, D_OUT), lambda i: (0, i, 0)),
            pl.BlockSpec((_BR, D_OUT), lambda i: (i, 0)),
            pl.BlockSpec((_BR, 1), lambda i: (i, 0)),
            pl.BlockSpec((1, D_OUT), lambda i: (0, 0)),
        ],
        out_specs=pl.BlockSpec((_BR, D_OUT), lambda i: (i, 0)),
        out_shape=jax.ShapeDtypeStruct((N, D_OUT), jnp.float32),
    )(s3, xs3, dis, b3.reshape(1, D_OUT))


def kernel(x, edge_index, W1, b1, W2, b2, W3, b3):
    src = edge_index[0].reshape(NW, NB, B)
    dst = edge_index[1].reshape(NW, NB, B)
    dst_deg = dst

    degp = _sc_degree(dst_deg)
    dis, xs1 = _tc_pre(degp, x)

    s1 = _sc_spmm(xs1, src, dst)
    xs2 = _tc_layer1(s1, xs1, dis, W1, b1)

    s2a = _sc_spmm(xs2[:, :D_IN], src, dst)
    s2b = _sc_spmm(xs2[:, D_IN:], src, dst)
    xs3 = _tc_layer2(s2a, s2b, xs2, dis, W2, b2, W3)

    s3 = _sc_spmm(xs3, src, dst)
    return _tc_final(s3, xs3, dis, b3)


# speedup vs baseline: 2.3715x; 1.0004x over previous
import functools

import jax
import jax.numpy as jnp
from jax import lax
from jax.experimental import pallas as pl
from jax.experimental.pallas import tpu as pltpu
from jax.experimental.pallas import tpu_sc as plsc

N = 10000
E = 320000
D_IN = 128
D_HID = 256
D_OUT = 128

NC = 2
NS = 16
NW = NC * NS
EPW = E // NW
B = 80
NB = EPW // B
NPAD = 10240
RPT = NPAD // NS

_mesh = plsc.VectorSubcoreMesh(core_axis_name="c", subcore_axis_name="s")


def _zero_stage(stage_v, rows, width):
    nv = width // 16

    def body(i, _):
        for j in range(nv):
            stage_v[i, pl.ds(j * 16, 16)] = jnp.zeros((16,), jnp.float32)
        return 0

    lax.fori_loop(0, rows, body, 0)


@functools.partial(
    pl.kernel,
    out_type=jax.ShapeDtypeStruct((NC, NPAD, 128), jnp.float32),
    mesh=_mesh,
    scratch_types=[
        pltpu.VMEM((NB, B), jnp.int32),
        pltpu.VMEM((B, 128), jnp.float32),
        pltpu.VMEM_SHARED((NPAD, 128), jnp.float32),
    ],
)
def _sc_degree(dst_hbm, out_hbm, dst_v, ones_v, acc_sh):
    c = lax.axis_index("c")
    s = lax.axis_index("s")
    wid = s * NC + c

    _zero_stage(ones_v, B, 128)
    for t in range(RPT // B):
        pltpu.sync_copy(ones_v, acc_sh.at[pl.ds(s * RPT + t * B, B)])
    plsc.subcore_barrier()

    def fill_ones(i, _):
        for j in range(8):
            ones_v[i, pl.ds(j * 16, 16)] = jnp.ones((16,), jnp.float32)
        return 0

    lax.fori_loop(0, B, fill_ones, 0)

    pltpu.sync_copy(dst_hbm.at[wid], dst_v)

    def body(j, _):
        pltpu.sync_copy(ones_v, acc_sh.at[dst_v.at[j]], add=True)
        return 0

    lax.fori_loop(0, NB, body, 0)
    plsc.subcore_barrier()

    for t in range(RPT // B):
        r0 = s * RPT + t * B
        pltpu.sync_copy(acc_sh.at[pl.ds(r0, B)], ones_v)
        pltpu.sync_copy(ones_v, out_hbm.at[c, pl.ds(r0, B)])


@functools.partial(
    pl.kernel,
    out_type=jax.ShapeDtypeStruct((NC, NPAD, 128), jnp.float32),
    mesh=_mesh,
    scratch_types=[
        pltpu.VMEM((NB, B), jnp.int32),
        pltpu.VMEM((NB, B), jnp.int32),
        pltpu.VMEM((B, 128), jnp.float32),
        pltpu.VMEM_SHARED((NPAD, 128), jnp.float32),
        pltpu.SemaphoreType.DMA,
    ],
)
def _sc_spmm(xs_hbm, src_hbm, dst_hbm, out_hbm,
             src_v, dst_v, rows_v, acc_sh, sem):
    c = lax.axis_index("c")
    s = lax.axis_index("s")
    wid = s * NC + c

    _zero_stage(rows_v, B, 128)
    for t in range(RPT // B):
        pltpu.sync_copy(rows_v, acc_sh.at[pl.ds(s * RPT + t * B, B)])
    plsc.subcore_barrier()

    pltpu.sync_copy(src_hbm.at[wid], src_v)
    pltpu.sync_copy(dst_hbm.at[wid], dst_v)

    def body(j, _):
        pltpu.async_copy(xs_hbm.at[src_v.at[j]], rows_v, sem).wait()
        pltpu.sync_copy(rows_v, acc_sh.at[dst_v.at[j]], add=True)
        return 0

    lax.fori_loop(0, NB, body, 0)
    plsc.subcore_barrier()

    for t in range(RPT // B):
        r0 = s * RPT + t * B
        pltpu.sync_copy(acc_sh.at[pl.ds(r0, B)], rows_v)
        pltpu.sync_copy(rows_v, out_hbm.at[c, pl.ds(r0, B)])


_BR = 400
_GRID = N // _BR


def _tc_pre_body(degp_ref, x_ref, dis_ref, xs1_ref):
    deg = degp_ref[0, :, 0:1] + degp_ref[1, :, 0:1] + 1.0
    dis = lax.rsqrt(deg)
    dis_ref[...] = dis
    xs1_ref[...] = x_ref[...] * dis


def _tc_pre(degp, x):
    return pl.pallas_call(
        _tc_pre_body,
        grid=(_GRID,),
        in_specs=[
            pl.BlockSpec((NC, _BR, 128), lambda i: (0, i, 0)),
            pl.BlockSpec((_BR, D_IN), lambda i: (i, 0)),
        ],
        out_specs=[
            pl.BlockSpec((_BR, 1), lambda i: (i, 0)),
            pl.BlockSpec((_BR, D_IN), lambda i: (i, 0)),
        ],
        out_shape=[
            jax.ShapeDtypeStruct((N, 1), jnp.float32),
            jax.ShapeDtypeStruct((N, D_IN), jnp.float32),
        ],
    )(degp, x)


def _tc_layer1_body(s1_ref, xs1_ref, dis_ref, w1_ref, b1_ref, xs2_ref):
    dis = dis_ref[...]
    agg = dis * (s1_ref[0] + s1_ref[1] + xs1_ref[...])
    y1 = jnp.maximum(
        jnp.dot(agg, w1_ref[...], preferred_element_type=jnp.float32)
        + b1_ref[...], 0.0)
    xs2_ref[...] = dis * y1


def _tc_layer1(s1, xs1, dis, W1, b1):
    return pl.pallas_call(
        _tc_layer1_body,
        grid=(_GRID,),
        in_specs=[
            pl.BlockSpec((NC, _BR, D_IN), lambda i: (0, i, 0)),
            pl.BlockSpec((_BR, D_IN), lambda i: (i, 0)),
            pl.BlockSpec((_BR, 1), lambda i: (i, 0)),
            pl.BlockSpec((D_IN, D_HID), lambda i: (0, 0)),
            pl.BlockSpec((1, D_HID), lambda i: (0, 0)),
        ],
        out_specs=pl.BlockSpec((_BR, D_HID), lambda i: (i, 0)),
        out_shape=jax.ShapeDtypeStruct((N, D_HID), jnp.float32),
    )(s1, xs1, dis, W1, b1.reshape(1, D_HID))


def _tc_layer2_body(s2a_ref, s2b_ref, xs2_ref, dis_ref, w2_ref, b2_ref,
                    w3_ref, xs3_ref):
    dis = dis_ref[...]
    xs2 = xs2_ref[...]
    agg_a = dis * (s2a_ref[0] + s2a_ref[1] + xs2[:, :D_IN])
    agg_b = dis * (s2b_ref[0] + s2b_ref[1] + xs2[:, D_IN:])
    agg = jnp.concatenate([agg_a, agg_b], axis=1)
    y2 = jnp.maximum(
        jnp.dot(agg, w2_ref[...], preferred_element_type=jnp.float32)
        + b2_ref[...], 0.0)
    z = jnp.dot(y2, w3_ref[...], preferred_element_type=jnp.float32)
    xs3_ref[...] = dis * z


def _tc_layer2(s2a, s2b, xs2, dis, W2, b2, W3):
    return pl.pallas_call(
        _tc_layer2_body,
        grid=(_GRID,),
        in_specs=[
            pl.BlockSpec((NC, _BR, D_IN), lambda i: (0, i, 0)),
            pl.BlockSpec((NC, _BR, D_IN), lambda i: (0, i, 0)),
            pl.BlockSpec((_BR, D_HID), lambda i: (i, 0)),
            pl.BlockSpec((_BR, 1), lambda i: (i, 0)),
            pl.BlockSpec((D_HID, D_HID), lambda i: (0, 0)),
            pl.BlockSpec((1, D_HID), lambda i: (0, 0)),
            pl.BlockSpec((D_HID, D_OUT), lambda i: (0, 0)),
        ],
        out_specs=pl.BlockSpec((_BR, D_OUT), lambda i: (i, 0)),
        out_shape=jax.ShapeDtypeStruct((N, D_OUT), jnp.float32),
    )(s2a, s2b, xs2, dis, W2, b2.reshape(1, D_HID), W3)


def _tc_final_body(s3_ref, xs3_ref, dis_ref, b3_ref, out_ref):
    dis = dis_ref[...]
    out_ref[...] = dis * (s3_ref[0] + s3_ref[1] + xs3_ref[...]) + b3_ref[...]


def _tc_final(s3, xs3, dis, b3):
    return pl.pallas_call(
        _tc_final_body,
        grid=(_GRID,),
        in_specs=[
            pl.BlockSpec((NC, _BR, D_OUT), lambda i: (0, i, 0)),
            pl.BlockSpec((_BR, D_OUT), lambda i: (i, 0)),
            pl.BlockSpec((_BR, 1), lambda i: (i, 0)),
            pl.BlockSpec((1, D_OUT), lambda i: (0, 0)),
        ],
        out_specs=pl.BlockSpec((_BR, D_OUT), lambda i: (i, 0)),
        out_shape=jax.ShapeDtypeStruct((N, D_OUT), jnp.float32),
    )(s3, xs3, dis, b3.reshape(1, D_OUT))


def kernel(x, edge_index, W1, b1, W2, b2, W3, b3):
    src = edge_index[0].reshape(NW, NB, B)
    dst = edge_index[1].reshape(NW, NB, B)
    dst_deg = dst

    degp = _sc_degree(dst_deg)
    dis, xs1 = _tc_pre(degp, x)

    s1 = _sc_spmm(xs1, src, dst)
    xs2 = _tc_layer1(s1, xs1, dis, W1, b1)

    s2a = _sc_spmm(xs2[:, :D_IN], src, dst)
    s2b = _sc_spmm(xs2[:, D_IN:], src, dst)
    xs3 = _tc_layer2(s2a, s2b, xs2, dis, W2, b2, W3)

    s3 = _sc_spmm(xs3, src, dst)
    return _tc_final(s3, xs3, dis, b3)
